# baseline (speedup 1.0000x reference)
import functools

import numpy as np
import jax
import jax.numpy as jnp
from jax import lax
from jax.experimental import pallas as pl
from jax.experimental.pallas import tpu as pltpu
from jax.experimental.pallas import tpu_sc as plsc

B, T, S = 4096, 10000, 128

NUM_CORES = 2
NUM_SUBCORES = 16
NUM_WORKERS = NUM_CORES * NUM_SUBCORES
ROWS_PER_W = B // NUM_WORKERS


def _subsample_taps():
    max_logspace = np.log10(T - 1)
    idx = np.round(np.logspace(0.0, max_logspace, S, endpoint=True), 1).astype(int)
    idx[0] = 0
    return idx.astype(np.int32)


_TAPS = _subsample_taps()
_FLAT_IDX = (np.arange(B, dtype=np.int64)[:, None] * T + _TAPS[None, :]).astype(
    np.int32
)


ELEMS_PER_W = ROWS_PER_W * S


def _sc_gather_body(xf_hbm, fidx_hbm, out_hbm, idx_v, data_v, sem):
    wid = lax.axis_index("s") * NUM_CORES + lax.axis_index("c")
    base = wid * ELEMS_PER_W
    pltpu.sync_copy(fidx_hbm.at[pl.ds(base, ELEMS_PER_W)], idx_v)
    pltpu.async_copy(xf_hbm.at[idx_v], data_v, sem).wait()
    pltpu.sync_copy(data_v, out_hbm.at[pl.ds(base, ELEMS_PER_W)])


_sc_gather = functools.partial(
    pl.kernel,
    mesh=plsc.VectorSubcoreMesh(core_axis_name="c", subcore_axis_name="s"),
    out_type=jax.ShapeDtypeStruct((B * S,), jnp.float32),
    scratch_types=[
        pltpu.VMEM((ELEMS_PER_W,), jnp.int32),
        pltpu.VMEM((ELEMS_PER_W,), jnp.float32),
        pltpu.SemaphoreType.DMA,
    ],
)(_sc_gather_body)


@jax.jit
def kernel(x):
    xf = x.reshape(-1)
    fidx = jnp.asarray(_FLAT_IDX.reshape(-1))
    return _sc_gather(xf, fidx).reshape(B, S)


# speedup vs baseline: 1.9668x; 1.9668x over previous
import numpy as np
import jax
import jax.numpy as jnp
from jax.experimental import pallas as pl
from jax.experimental.pallas import tpu as pltpu

B, T, S = 4096, 10000, 128
SLAB_TILES = 4
SLAB = SLAB_TILES * 128
BR = 1024
NBLK = B // BR


def _subsample_taps():
    max_logspace = np.log10(T - 1)
    idx = np.round(np.logspace(0.0, max_logspace, S, endpoint=True), 1).astype(int)
    idx[0] = 0
    return idx.astype(np.int32)


def _build_plan():
    taps = _subsample_taps()
    high_tiles = sorted(set(int(t) // 128 for t in taps if t >= SLAB))
    w_slab = np.zeros((SLAB, S), np.float32)
    w_tiles = [np.zeros((128, S), np.float32) for _ in high_tiles]
    tile_pos = {c: k for k, c in enumerate(high_tiles)}
    for j, t in enumerate(taps):
        t = int(t)
        if t < SLAB:
            w_slab[t, j] = 1.0
        else:
            w_tiles[tile_pos[t // 128]][t % 128, j] = 1.0
    return high_tiles, w_slab, w_tiles


_HIGH_TILES, _W_SLAB, _W_TILES = _build_plan()
NT = len(_HIGH_TILES)


def _tc_body(*refs):
    slab_ref = refs[0]
    tile_refs = refs[1 : 1 + NT]
    w_slab_ref = refs[1 + NT]
    w_tile_ref = refs[2 + NT]
    out_ref = refs[3 + NT]
    acc = jnp.dot(slab_ref[...], w_slab_ref[...], preferred_element_type=jnp.float32)
    for k in range(NT):
        acc += jnp.dot(
            tile_refs[k][...], w_tile_ref[k], preferred_element_type=jnp.float32
        )
    out_ref[...] = acc


def _make_tile_spec(col_tile):
    return pl.BlockSpec((BR, 128), lambda i, c=col_tile: (i, c))


_tc_gather = pl.pallas_call(
    _tc_body,
    grid=(NBLK,),
    in_specs=(
        [pl.BlockSpec((BR, SLAB), lambda i: (i, 0))]
        + [_make_tile_spec(c) for c in _HIGH_TILES]
        + [
            pl.BlockSpec((SLAB, S), lambda i: (0, 0)),
            pl.BlockSpec((NT, 128, S), lambda i: (0, 0, 0)),
        ]
    ),
    out_specs=pl.BlockSpec((BR, S), lambda i: (i, 0)),
    out_shape=jax.ShapeDtypeStruct((B, S), jnp.float32),
    compiler_params=pltpu.CompilerParams(
        dimension_semantics=("arbitrary",),
    ),
)


@jax.jit
def kernel(x):
    x2d = jnp.squeeze(x, axis=1)
    w_slab = jnp.asarray(_W_SLAB)
    w_tiles = jnp.asarray(np.stack(_W_TILES))
    return _tc_gather(x2d, *([x2d] * NT), w_slab, w_tiles)
